# baseline (speedup 1.0000x reference)
import functools

import jax
import jax.numpy as jnp
from jax import lax
from jax.experimental import pallas as pl
from jax.experimental.pallas import tpu as pltpu
from jax.experimental.pallas import tpu_sc as plsc

B, S = 4, 8192
N = B * S
DOUT = 192
NV = DOUT // 16
MAX2D = 1024
EPS = 1e-12

NC, NS = 2, 16
NW = NC * NS
TPW = N // NW
C = 32
NCHUNK = TPW // C


def _project_body(x_ref, y_ref, h_ref, w_ref, W_ref, b_ref, out_ref):
    f32 = jnp.float32
    out_ref[0 * MAX2D:1 * MAX2D, :] = jnp.dot(
        x_ref[...], W_ref[0:128, :], preferred_element_type=f32)
    out_ref[1 * MAX2D:2 * MAX2D, :] = jnp.dot(
        y_ref[...], W_ref[128:256, :], preferred_element_type=f32)
    out_ref[2 * MAX2D:3 * MAX2D, :] = jnp.dot(
        x_ref[...], W_ref[256:384, :], preferred_element_type=f32)
    out_ref[3 * MAX2D:4 * MAX2D, :] = jnp.dot(
        y_ref[...], W_ref[384:512, :], preferred_element_type=f32)
    out_ref[4 * MAX2D:5 * MAX2D, :] = jnp.dot(
        h_ref[...], W_ref[512:640, :], preferred_element_type=f32) + b_ref[...]
    out_ref[5 * MAX2D:6 * MAX2D, :] = jnp.dot(
        w_ref[...], W_ref[640:768, :], preferred_element_type=f32)


_project = pl.pallas_call(
    _project_body,
    out_shape=jax.ShapeDtypeStruct((6 * MAX2D, DOUT), jnp.float32),
)

_sc_mesh = plsc.VectorSubcoreMesh(
    core_axis_name="c", subcore_axis_name="s", num_cores=NC, num_subcores=NS)


@functools.partial(
    pl.kernel,
    out_type=jax.ShapeDtypeStruct((N, DOUT), jnp.float32),
    mesh=_sc_mesh,
    compiler_params=pltpu.CompilerParams(
        needs_layout_passes=False, use_tc_tiling_on_sc=False),
    scratch_types=[
        pltpu.VMEM((TPW,), jnp.int32),
        pltpu.VMEM((TPW,), jnp.int32),
        pltpu.VMEM((TPW,), jnp.int32),
        pltpu.VMEM((TPW,), jnp.int32),
        pltpu.VMEM((TPW,), jnp.int32),
        pltpu.VMEM((TPW,), jnp.int32),
        pltpu.VMEM((TPW,), jnp.int32),
        pltpu.VMEM((TPW,), jnp.int32),
        pltpu.VMEM((TPW,), jnp.int32),
        pltpu.VMEM((TPW,), jnp.int32),
        pltpu.VMEM((TPW,), jnp.int32),
        [pltpu.VMEM((C, DOUT), jnp.float32)] * 7,
        [pltpu.VMEM((C, DOUT), jnp.float32)] * 7,
        pltpu.VMEM((C, DOUT), jnp.float32),
        pltpu.VMEM((C, DOUT), jnp.float32),
        pltpu.VMEM((DOUT,), jnp.float32),
        pltpu.VMEM((DOUT,), jnp.float32),
        pltpu.SemaphoreType.DMA,
        pltpu.SemaphoreType.DMA,
    ],
)
def _lookup(ptab_h, box_h, x0_h, y0_h, x1_h, y1_h, pos_h, gam_h, bet_h,
            out_h,
            x0v, y0v, x1v, y1v, pv, i0, i1, i2, i3, i4, i5,
            gb0, gb1, ov0, ov1, gam, bet, sem0, sem1):
    wid = lax.axis_index("s") * NC + lax.axis_index("c")
    base = wid * TPW

    pltpu.sync_copy(x0_h.at[pl.ds(base, TPW)], x0v)
    pltpu.sync_copy(y0_h.at[pl.ds(base, TPW)], y0v)
    pltpu.sync_copy(x1_h.at[pl.ds(base, TPW)], x1v)
    pltpu.sync_copy(y1_h.at[pl.ds(base, TPW)], y1v)
    pltpu.sync_copy(pos_h.at[pl.ds(base, TPW)], pv)
    pltpu.sync_copy(gam_h, gam)
    pltpu.sync_copy(bet_h, bet)

    def idx_body(v, carry):
        sl = pl.ds(v * 16, 16)
        a0 = x0v[sl]
        c0 = y0v[sl]
        a1 = x1v[sl]
        c1 = y1v[sl]
        i0[sl] = a0
        i1[sl] = c0 + 1 * MAX2D
        i2[sl] = a1 + 2 * MAX2D
        i3[sl] = c1 + 3 * MAX2D
        i4[sl] = (c1 - c0) + 4 * MAX2D
        i5[sl] = (a1 - a0) + 5 * MAX2D
        return carry

    lax.fori_loop(0, TPW // 16, idx_body, 0, unroll=False)

    gl = [gam[pl.ds(c * 16, 16)] for c in range(NV)]
    bl = [bet[pl.ds(c * 16, 16)] for c in range(NV)]

    gsets = (gb0, gb1)
    osets = (ov0, ov1)
    sems = (sem0, sem1)

    def fire(k, par):
        off = k * C
        gb = gsets[par]
        sm = sems[par]
        pltpu.async_copy(ptab_h.at[i0.at[pl.ds(off, C)]], gb[0], sm)
        pltpu.async_copy(ptab_h.at[i1.at[pl.ds(off, C)]], gb[1], sm)
        pltpu.async_copy(ptab_h.at[i2.at[pl.ds(off, C)]], gb[2], sm)
        pltpu.async_copy(ptab_h.at[i3.at[pl.ds(off, C)]], gb[3], sm)
        pltpu.async_copy(ptab_h.at[i4.at[pl.ds(off, C)]], gb[4], sm)
        pltpu.async_copy(ptab_h.at[i5.at[pl.ds(off, C)]], gb[5], sm)
        pltpu.async_copy(box_h.at[pv.at[pl.ds(off, C)]], gb[6], sm)

    def drain(par):
        gb = gsets[par]
        sm = sems[par]
        for j in range(7):
            pltpu.make_async_copy(ptab_h.at[pl.ds(0, C)], gb[j], sm).wait()

    def compute(k, par):
        gb = gsets[par]
        ov = osets[par]

        def token(t, tc):
            s = jnp.zeros((16,), jnp.float32)
            q = jnp.zeros((16,), jnp.float32)
            vs = []
            for c in range(NV):
                sl = pl.ds(c * 16, 16)
                v = (gb[0][t, sl] + gb[1][t, sl] + gb[2][t, sl]
                     + gb[3][t, sl] + gb[4][t, sl] + gb[5][t, sl]
                     + gb[6][t, sl])
                vs.append(v)
                s = s + v
                q = q + v * v
            tot = plsc.cumsum(s)[15]
            totq = plsc.cumsum(q)[15]
            mu = tot * (1.0 / DOUT)
            var = totq * (1.0 / DOUT) - mu * mu
            xv = jnp.full((16,), var + EPS, dtype=jnp.float32)
            iv = lax.bitcast_convert_type(xv, jnp.int32)
            iv = jnp.int32(0x5F3759DF) - (iv >> 1)
            y = lax.bitcast_convert_type(iv, jnp.float32)
            for _ in range(3):
                y = y * (1.5 - 0.5 * xv * y * y)
            muv = jnp.full((16,), mu, dtype=jnp.float32)
            for c in range(NV):
                sl = pl.ds(c * 16, 16)
                ov[t, sl] = (vs[c] - muv) * y * gl[c] + bl[c]
            return tc

        lax.fori_loop(0, C, token, 0, unroll=4)
        pltpu.sync_copy(ov, out_h.at[pl.ds(base + k * C, C)])

    fire(0, 0)

    def pipe(j, carry):
        k = 2 * j
        fire(k + 1, 1)
        drain(0)
        compute(k, 0)

        @pl.when(k + 2 < NCHUNK)
        def _():
            fire(k + 2, 0)

        drain(1)
        compute(k + 1, 1)
        return carry

    lax.fori_loop(0, NCHUNK // 2, pipe, 0, unroll=False)


def kernel(bbox, position_ids, x_tab, y_tab, h_tab, w_tab, box_tab, W, b,
           gamma, beta):
    bb = bbox.reshape(N, 4).astype(jnp.int32)
    x0 = bb[:, 0]
    y0 = bb[:, 1]
    x1 = bb[:, 2]
    y1 = bb[:, 3]
    pos = position_ids.reshape(N).astype(jnp.int32)
    ptab = _project(x_tab, y_tab, h_tab, w_tab, W, b.reshape(1, DOUT))
    out = _lookup(ptab, box_tab, x0, y0, x1, y1, pos, gamma, beta)
    return out.reshape(B, S, DOUT)


# speedup vs baseline: 1.0539x; 1.0539x over previous
import functools

import jax
import jax.numpy as jnp
from jax import lax
from jax.experimental import pallas as pl
from jax.experimental.pallas import tpu as pltpu
from jax.experimental.pallas import tpu_sc as plsc

B, S = 4, 8192
N = B * S
DOUT = 192
NV = DOUT // 16
MAX2D = 1024
EPS = 1e-12

NC, NS = 2, 16
NW = NC * NS
TPW = N // NW
C = 32
NCHUNK = TPW // C


def _project_body(x_ref, y_ref, h_ref, w_ref, W_ref, b_ref, out_ref):
    f32 = jnp.float32
    out_ref[0 * MAX2D:1 * MAX2D, :] = jnp.dot(
        x_ref[...], W_ref[0:128, :], preferred_element_type=f32)
    out_ref[1 * MAX2D:2 * MAX2D, :] = jnp.dot(
        y_ref[...], W_ref[128:256, :], preferred_element_type=f32)
    out_ref[2 * MAX2D:3 * MAX2D, :] = jnp.dot(
        x_ref[...], W_ref[256:384, :], preferred_element_type=f32)
    out_ref[3 * MAX2D:4 * MAX2D, :] = jnp.dot(
        y_ref[...], W_ref[384:512, :], preferred_element_type=f32)
    out_ref[4 * MAX2D:5 * MAX2D, :] = jnp.dot(
        h_ref[...], W_ref[512:640, :], preferred_element_type=f32) + b_ref[...]
    out_ref[5 * MAX2D:6 * MAX2D, :] = jnp.dot(
        w_ref[...], W_ref[640:768, :], preferred_element_type=f32)


_project = pl.pallas_call(
    _project_body,
    out_shape=jax.ShapeDtypeStruct((6 * MAX2D, DOUT), jnp.float32),
)

_sc_mesh = plsc.VectorSubcoreMesh(
    core_axis_name="c", subcore_axis_name="s", num_cores=NC, num_subcores=NS)


@functools.partial(
    pl.kernel,
    out_type=jax.ShapeDtypeStruct((N, DOUT), jnp.float32),
    mesh=_sc_mesh,
    compiler_params=pltpu.CompilerParams(
        needs_layout_passes=False, use_tc_tiling_on_sc=False),
    scratch_types=[
        pltpu.VMEM((TPW,), jnp.int32),
        pltpu.VMEM((TPW,), jnp.int32),
        pltpu.VMEM((TPW,), jnp.int32),
        pltpu.VMEM((TPW,), jnp.int32),
        pltpu.VMEM((TPW,), jnp.int32),
        pltpu.VMEM((TPW,), jnp.int32),
        pltpu.VMEM((TPW,), jnp.int32),
        pltpu.VMEM((TPW,), jnp.int32),
        pltpu.VMEM((TPW,), jnp.int32),
        pltpu.VMEM((TPW,), jnp.int32),
        pltpu.VMEM((TPW,), jnp.int32),
        [pltpu.VMEM((C, DOUT), jnp.float32)] * 7,
        [pltpu.VMEM((C, DOUT), jnp.float32)] * 7,
        pltpu.VMEM((C, DOUT), jnp.float32),
        pltpu.VMEM((C, DOUT), jnp.float32),
        pltpu.VMEM((DOUT,), jnp.float32),
        pltpu.VMEM((DOUT,), jnp.float32),
        pltpu.SemaphoreType.DMA,
        pltpu.SemaphoreType.DMA,
    ],
)
def _lookup(ptab_h, box_h, x0_h, y0_h, x1_h, y1_h, pos_h, gam_h, bet_h,
            out_h,
            x0v, y0v, x1v, y1v, pv, i0, i1, i2, i3, i4, i5,
            gb0, gb1, ov0, ov1, gam, bet, sem0, sem1):
    wid = lax.axis_index("s") * NC + lax.axis_index("c")
    base = wid * TPW

    pltpu.sync_copy(x0_h.at[pl.ds(base, TPW)], x0v)
    pltpu.sync_copy(y0_h.at[pl.ds(base, TPW)], y0v)
    pltpu.sync_copy(x1_h.at[pl.ds(base, TPW)], x1v)
    pltpu.sync_copy(y1_h.at[pl.ds(base, TPW)], y1v)
    pltpu.sync_copy(pos_h.at[pl.ds(base, TPW)], pv)
    pltpu.sync_copy(gam_h, gam)
    pltpu.sync_copy(bet_h, bet)

    def idx_body(v, carry):
        sl = pl.ds(v * 16, 16)
        a0 = x0v[sl]
        c0 = y0v[sl]
        a1 = x1v[sl]
        c1 = y1v[sl]
        i0[sl] = a0
        i1[sl] = c0 + 1 * MAX2D
        i2[sl] = a1 + 2 * MAX2D
        i3[sl] = c1 + 3 * MAX2D
        i4[sl] = (c1 - c0) + 4 * MAX2D
        i5[sl] = (a1 - a0) + 5 * MAX2D
        return carry

    lax.fori_loop(0, TPW // 16, idx_body, 0, unroll=False)

    gl = [gam[pl.ds(c * 16, 16)] for c in range(NV)]
    bl = [bet[pl.ds(c * 16, 16)] for c in range(NV)]

    gsets = (gb0, gb1)
    osets = (ov0, ov1)
    sems = (sem0, sem1)

    def fire(k, par):
        off = k * C
        gb = gsets[par]
        sm = sems[par]
        pltpu.async_copy(ptab_h.at[i0.at[pl.ds(off, C)]], gb[0], sm)
        pltpu.async_copy(ptab_h.at[i1.at[pl.ds(off, C)]], gb[1], sm)
        pltpu.async_copy(ptab_h.at[i2.at[pl.ds(off, C)]], gb[2], sm)
        pltpu.async_copy(ptab_h.at[i3.at[pl.ds(off, C)]], gb[3], sm)
        pltpu.async_copy(ptab_h.at[i4.at[pl.ds(off, C)]], gb[4], sm)
        pltpu.async_copy(ptab_h.at[i5.at[pl.ds(off, C)]], gb[5], sm)
        pltpu.async_copy(box_h.at[pv.at[pl.ds(off, C)]], gb[6], sm)

    def drain(par):
        gb = gsets[par]
        sm = sems[par]
        for j in range(7):
            pltpu.make_async_copy(ptab_h.at[pl.ds(0, C)], gb[j], sm).wait()

    def compute(k, par):
        gb = gsets[par]
        ov = osets[par]

        def token(t, tc):
            s = jnp.zeros((16,), jnp.float32)
            q = jnp.zeros((16,), jnp.float32)
            vs = []
            for c in range(NV):
                sl = pl.ds(c * 16, 16)
                v = (gb[0][t, sl] + gb[1][t, sl] + gb[2][t, sl]
                     + gb[3][t, sl] + gb[4][t, sl] + gb[5][t, sl]
                     + gb[6][t, sl])
                vs.append(v)
                s = s + v
                q = q + v * v
            tot = plsc.cumsum(s)[15]
            totq = plsc.cumsum(q)[15]
            mu = tot * (1.0 / DOUT)
            var = totq * (1.0 / DOUT) - mu * mu
            xv = jnp.full((16,), var + EPS, dtype=jnp.float32)
            iv = lax.bitcast_convert_type(xv, jnp.int32)
            iv = jnp.int32(0x5F3759DF) - (iv >> 1)
            y = lax.bitcast_convert_type(iv, jnp.float32)
            for _ in range(3):
                y = y * (1.5 - 0.5 * xv * y * y)
            muv = jnp.full((16,), mu, dtype=jnp.float32)
            for c in range(NV):
                sl = pl.ds(c * 16, 16)
                ov[t, sl] = (vs[c] - muv) * y * gl[c] + bl[c]
            return tc

        lax.fori_loop(0, C, token, 0, unroll=2)
        pltpu.sync_copy(ov, out_h.at[pl.ds(base + k * C, C)])

    fire(0, 0)

    def pipe(j, carry):
        k = 2 * j
        fire(k + 1, 1)
        drain(0)
        compute(k, 0)

        @pl.when(k + 2 < NCHUNK)
        def _():
            fire(k + 2, 0)

        drain(1)
        compute(k + 1, 1)
        return carry

    lax.fori_loop(0, NCHUNK // 2, pipe, 0, unroll=False)


def kernel(bbox, position_ids, x_tab, y_tab, h_tab, w_tab, box_tab, W, b,
           gamma, beta):
    bb = bbox.reshape(N, 4).astype(jnp.int32)
    x0 = bb[:, 0]
    y0 = bb[:, 1]
    x1 = bb[:, 2]
    y1 = bb[:, 3]
    pos = position_ids.reshape(N).astype(jnp.int32)
    ptab = _project(x_tab, y_tab, h_tab, w_tab, W, b.reshape(1, DOUT))
    out = _lookup(ptab, box_tab, x0, y0, x1, y1, pos, gamma, beta)
    return out.reshape(B, S, DOUT)


# speedup vs baseline: 1.1618x; 1.1023x over previous
import functools

import jax
import jax.numpy as jnp
from jax import lax
from jax.experimental import pallas as pl
from jax.experimental.pallas import tpu as pltpu
from jax.experimental.pallas import tpu_sc as plsc

B, S = 4, 8192
N = B * S
DOUT = 192
NV = DOUT // 16
MAX2D = 1024
EPS = 1e-12

NC, NS = 2, 16
NW = NC * NS
TPW = N // NW
C = 32
NCHUNK = TPW // C


def _project_body(x_ref, y_ref, h_ref, w_ref, W_ref, b_ref, out_ref):
    f32 = jnp.float32
    out_ref[0 * MAX2D:1 * MAX2D, :] = jnp.dot(
        x_ref[...], W_ref[0:128, :], preferred_element_type=f32)
    out_ref[1 * MAX2D:2 * MAX2D, :] = jnp.dot(
        y_ref[...], W_ref[128:256, :], preferred_element_type=f32)
    out_ref[2 * MAX2D:3 * MAX2D, :] = jnp.dot(
        x_ref[...], W_ref[256:384, :], preferred_element_type=f32)
    out_ref[3 * MAX2D:4 * MAX2D, :] = jnp.dot(
        y_ref[...], W_ref[384:512, :], preferred_element_type=f32)
    out_ref[4 * MAX2D:5 * MAX2D, :] = jnp.dot(
        h_ref[...], W_ref[512:640, :], preferred_element_type=f32) + b_ref[...]
    out_ref[5 * MAX2D:6 * MAX2D, :] = jnp.dot(
        w_ref[...], W_ref[640:768, :], preferred_element_type=f32)


_project = pl.pallas_call(
    _project_body,
    out_shape=jax.ShapeDtypeStruct((6 * MAX2D, DOUT), jnp.float32),
)

_sc_mesh = plsc.VectorSubcoreMesh(
    core_axis_name="c", subcore_axis_name="s", num_cores=NC, num_subcores=NS)


@functools.partial(
    pl.kernel,
    out_type=jax.ShapeDtypeStruct((N, DOUT), jnp.float32),
    mesh=_sc_mesh,
    compiler_params=pltpu.CompilerParams(
        needs_layout_passes=False, use_tc_tiling_on_sc=False),
    scratch_types=[
        pltpu.VMEM((TPW,), jnp.int32),
        pltpu.VMEM((TPW,), jnp.int32),
        pltpu.VMEM((TPW,), jnp.int32),
        pltpu.VMEM((TPW,), jnp.int32),
        pltpu.VMEM((TPW,), jnp.int32),
        pltpu.VMEM((TPW,), jnp.int32),
        pltpu.VMEM((TPW,), jnp.int32),
        pltpu.VMEM((TPW,), jnp.int32),
        pltpu.VMEM((TPW,), jnp.int32),
        pltpu.VMEM((TPW,), jnp.int32),
        pltpu.VMEM((TPW,), jnp.int32),
        [pltpu.VMEM((C, DOUT), jnp.float32)] * 7,
        [pltpu.VMEM((C, DOUT), jnp.float32)] * 7,
        pltpu.VMEM((C, DOUT), jnp.float32),
        pltpu.VMEM((C, DOUT), jnp.float32),
        pltpu.VMEM((DOUT,), jnp.float32),
        pltpu.VMEM((DOUT,), jnp.float32),
        pltpu.VMEM((16, 16), jnp.float32),
        pltpu.VMEM((16, 16), jnp.float32),
        pltpu.VMEM((16,), jnp.float32),
        pltpu.VMEM((16,), jnp.float32),
        pltpu.SemaphoreType.DMA,
        pltpu.SemaphoreType.DMA,
    ],
)
def _lookup(ptab_h, box_h, x0_h, y0_h, x1_h, y1_h, pos_h, gam_h, bet_h,
            out_h,
            x0v, y0v, x1v, y1v, pv, i0, i1, i2, i3, i4, i5,
            gb0, gb1, ov0, ov1, gam, bet, sb, qb, mb, rb, sem0, sem1):
    wid = lax.axis_index("s") * NC + lax.axis_index("c")
    base = wid * TPW

    pltpu.sync_copy(x0_h.at[pl.ds(base, TPW)], x0v)
    pltpu.sync_copy(y0_h.at[pl.ds(base, TPW)], y0v)
    pltpu.sync_copy(x1_h.at[pl.ds(base, TPW)], x1v)
    pltpu.sync_copy(y1_h.at[pl.ds(base, TPW)], y1v)
    pltpu.sync_copy(pos_h.at[pl.ds(base, TPW)], pv)
    pltpu.sync_copy(gam_h, gam)
    pltpu.sync_copy(bet_h, bet)

    def idx_body(v, carry):
        sl = pl.ds(v * 16, 16)
        a0 = x0v[sl]
        c0 = y0v[sl]
        a1 = x1v[sl]
        c1 = y1v[sl]
        i0[sl] = a0
        i1[sl] = c0 + 1 * MAX2D
        i2[sl] = a1 + 2 * MAX2D
        i3[sl] = c1 + 3 * MAX2D
        i4[sl] = (c1 - c0) + 4 * MAX2D
        i5[sl] = (a1 - a0) + 5 * MAX2D
        return carry

    lax.fori_loop(0, TPW // 16, idx_body, 0, unroll=False)

    gl = [gam[pl.ds(c * 16, 16)] for c in range(NV)]
    bl = [bet[pl.ds(c * 16, 16)] for c in range(NV)]

    gsets = (gb0, gb1)
    osets = (ov0, ov1)
    sems = (sem0, sem1)

    def fire(k, par):
        off = k * C
        gb = gsets[par]
        sm = sems[par]
        pltpu.async_copy(ptab_h.at[i0.at[pl.ds(off, C)]], gb[0], sm)
        pltpu.async_copy(ptab_h.at[i1.at[pl.ds(off, C)]], gb[1], sm)
        pltpu.async_copy(ptab_h.at[i2.at[pl.ds(off, C)]], gb[2], sm)
        pltpu.async_copy(ptab_h.at[i3.at[pl.ds(off, C)]], gb[3], sm)
        pltpu.async_copy(ptab_h.at[i4.at[pl.ds(off, C)]], gb[4], sm)
        pltpu.async_copy(ptab_h.at[i5.at[pl.ds(off, C)]], gb[5], sm)
        pltpu.async_copy(box_h.at[pv.at[pl.ds(off, C)]], gb[6], sm)

    def drain(par):
        gb = gsets[par]
        sm = sems[par]
        for j in range(7):
            pltpu.make_async_copy(ptab_h.at[pl.ds(0, C)], gb[j], sm).wait()

    lanes = jnp.arange(16, dtype=jnp.int32)

    def compute(k, par):
        gb = gsets[par]
        ov = osets[par]

        def token1(i, tc, t0=0):
            t = t0 + i
            s = jnp.zeros((16,), jnp.float32)
            q = jnp.zeros((16,), jnp.float32)
            for c in range(NV):
                sl = pl.ds(c * 16, 16)
                v = (gb[0][t, sl] + gb[1][t, sl] + gb[2][t, sl]
                     + gb[3][t, sl] + gb[4][t, sl] + gb[5][t, sl]
                     + gb[6][t, sl])
                ov[t, sl] = v
                s = s + v
                q = q + v * v
            sb[i, :] = s
            qb[i, :] = q
            return tc

        def token2(i, tc, t0=0):
            t = t0 + i
            iv16 = jnp.full((16,), i, jnp.int32)
            muv = plsc.load_gather(mb, [iv16])
            rv = plsc.load_gather(rb, [iv16])
            for c in range(NV):
                sl = pl.ds(c * 16, 16)
                ov[t, sl] = (ov[t, sl] - muv) * rv * gl[c] + bl[c]
            return tc

        for g in range(C // 16):
            t0 = g * 16
            lax.fori_loop(0, 16, functools.partial(token1, t0=t0), 0,
                          unroll=False)
            tot = jnp.zeros((16,), jnp.float32)
            totq = jnp.zeros((16,), jnp.float32)
            for l in range(16):
                cl = jnp.full((16,), l, jnp.int32)
                tot = tot + plsc.load_gather(sb, [lanes, cl])
                totq = totq + plsc.load_gather(qb, [lanes, cl])
            mu = tot * (1.0 / DOUT)
            var = totq * (1.0 / DOUT) - mu * mu
            xv = var + EPS
            iv = lax.bitcast_convert_type(xv, jnp.int32)
            iv = jnp.int32(0x5F3759DF) - (iv >> 1)
            y = lax.bitcast_convert_type(iv, jnp.float32)
            for _ in range(3):
                y = y * (1.5 - 0.5 * xv * y * y)
            mb[:] = mu
            rb[:] = y
            lax.fori_loop(0, 16, functools.partial(token2, t0=t0), 0,
                          unroll=False)

        pltpu.sync_copy(ov, out_h.at[pl.ds(base + k * C, C)])

    fire(0, 0)

    def pipe(j, carry):
        k = 2 * j
        fire(k + 1, 1)
        drain(0)
        compute(k, 0)

        @pl.when(k + 2 < NCHUNK)
        def _():
            fire(k + 2, 0)

        drain(1)
        compute(k + 1, 1)
        return carry

    lax.fori_loop(0, NCHUNK // 2, pipe, 0, unroll=False)


def kernel(bbox, position_ids, x_tab, y_tab, h_tab, w_tab, box_tab, W, b,
           gamma, beta):
    bb = bbox.reshape(N, 4).astype(jnp.int32)
    x0 = bb[:, 0]
    y0 = bb[:, 1]
    x1 = bb[:, 2]
    y1 = bb[:, 3]
    pos = position_ids.reshape(N).astype(jnp.int32)
    ptab = _project(x_tab, y_tab, h_tab, w_tab, W, b.reshape(1, DOUT))
    out = _lookup(ptab, box_tab, x0, y0, x1, y1, pos, gamma, beta)
    return out.reshape(B, S, DOUT)


# speedup vs baseline: 1.2395x; 1.0669x over previous
import functools

import jax
import jax.numpy as jnp
from jax import lax
from jax.experimental import pallas as pl
from jax.experimental.pallas import tpu as pltpu
from jax.experimental.pallas import tpu_sc as plsc

B, S = 4, 8192
N = B * S
DOUT = 192
NV = DOUT // 16
MAX2D = 1024
EPS = 1e-12

NC, NS = 2, 16
NW = NC * NS
TPW = N // NW
C = 32
NCHUNK = TPW // C


def _project_body(x_ref, y_ref, h_ref, w_ref, W_ref, b_ref, out_ref):
    f32 = jnp.float32
    out_ref[0 * MAX2D:1 * MAX2D, :] = jnp.dot(
        x_ref[...], W_ref[0:128, :], preferred_element_type=f32)
    out_ref[1 * MAX2D:2 * MAX2D, :] = jnp.dot(
        y_ref[...], W_ref[128:256, :], preferred_element_type=f32)
    out_ref[2 * MAX2D:3 * MAX2D, :] = jnp.dot(
        x_ref[...], W_ref[256:384, :], preferred_element_type=f32)
    out_ref[3 * MAX2D:4 * MAX2D, :] = jnp.dot(
        y_ref[...], W_ref[384:512, :], preferred_element_type=f32)
    out_ref[4 * MAX2D:5 * MAX2D, :] = jnp.dot(
        h_ref[...], W_ref[512:640, :], preferred_element_type=f32) + b_ref[...]
    out_ref[5 * MAX2D:6 * MAX2D, :] = jnp.dot(
        w_ref[...], W_ref[640:768, :], preferred_element_type=f32)


_project = pl.pallas_call(
    _project_body,
    out_shape=jax.ShapeDtypeStruct((6 * MAX2D, DOUT), jnp.float32),
)

_sc_mesh = plsc.VectorSubcoreMesh(
    core_axis_name="c", subcore_axis_name="s", num_cores=NC, num_subcores=NS)


@functools.partial(
    pl.kernel,
    out_type=jax.ShapeDtypeStruct((N, DOUT), jnp.float32),
    mesh=_sc_mesh,
    compiler_params=pltpu.CompilerParams(
        needs_layout_passes=False, use_tc_tiling_on_sc=False),
    scratch_types=[
        pltpu.VMEM((TPW,), jnp.int32),
        pltpu.VMEM((TPW,), jnp.int32),
        pltpu.VMEM((TPW,), jnp.int32),
        pltpu.VMEM((TPW,), jnp.int32),
        pltpu.VMEM((TPW,), jnp.int32),
        pltpu.VMEM((TPW,), jnp.int32),
        pltpu.VMEM((TPW,), jnp.int32),
        pltpu.VMEM((TPW,), jnp.int32),
        pltpu.VMEM((TPW,), jnp.int32),
        pltpu.VMEM((TPW,), jnp.int32),
        pltpu.VMEM((TPW,), jnp.int32),
        [pltpu.VMEM((C, DOUT), jnp.float32)] * 7,
        [pltpu.VMEM((C, DOUT), jnp.float32)] * 7,
        pltpu.VMEM((C, DOUT), jnp.float32),
        pltpu.VMEM((C, DOUT), jnp.float32),
        pltpu.VMEM((DOUT,), jnp.float32),
        pltpu.VMEM((DOUT,), jnp.float32),
        pltpu.SemaphoreType.DMA,
        pltpu.SemaphoreType.DMA,
        pltpu.SemaphoreType.DMA,
        pltpu.SemaphoreType.DMA,
    ],
)
def _lookup(ptab_h, box_h, x0_h, y0_h, x1_h, y1_h, pos_h, gam_h, bet_h,
            out_h,
            x0v, y0v, x1v, y1v, pv, i0, i1, i2, i3, i4, i5,
            gb0, gb1, ov0, ov1, gam, bet, sem0, sem1, osem0, osem1):
    wid = lax.axis_index("s") * NC + lax.axis_index("c")
    base = wid * TPW

    pltpu.sync_copy(x0_h.at[pl.ds(base, TPW)], x0v)
    pltpu.sync_copy(y0_h.at[pl.ds(base, TPW)], y0v)
    pltpu.sync_copy(x1_h.at[pl.ds(base, TPW)], x1v)
    pltpu.sync_copy(y1_h.at[pl.ds(base, TPW)], y1v)
    pltpu.sync_copy(pos_h.at[pl.ds(base, TPW)], pv)
    pltpu.sync_copy(gam_h, gam)
    pltpu.sync_copy(bet_h, bet)

    def idx_body(v, carry):
        sl = pl.ds(v * 16, 16)
        a0 = x0v[sl]
        c0 = y0v[sl]
        a1 = x1v[sl]
        c1 = y1v[sl]
        i0[sl] = a0
        i1[sl] = c0 + 1 * MAX2D
        i2[sl] = a1 + 2 * MAX2D
        i3[sl] = c1 + 3 * MAX2D
        i4[sl] = (c1 - c0) + 4 * MAX2D
        i5[sl] = (a1 - a0) + 5 * MAX2D
        return carry

    lax.fori_loop(0, TPW // 16, idx_body, 0, unroll=False)

    gl = [gam[pl.ds(c * 16, 16)] for c in range(NV)]
    bl = [bet[pl.ds(c * 16, 16)] for c in range(NV)]

    gsets = (gb0, gb1)
    osets = (ov0, ov1)
    sems = (sem0, sem1)
    osems = (osem0, osem1)

    def fire(k, par):
        off = k * C
        gb = gsets[par]
        sm = sems[par]
        pltpu.async_copy(ptab_h.at[i0.at[pl.ds(off, C)]], gb[0], sm)
        pltpu.async_copy(ptab_h.at[i1.at[pl.ds(off, C)]], gb[1], sm)
        pltpu.async_copy(ptab_h.at[i2.at[pl.ds(off, C)]], gb[2], sm)
        pltpu.async_copy(ptab_h.at[i3.at[pl.ds(off, C)]], gb[3], sm)
        pltpu.async_copy(ptab_h.at[i4.at[pl.ds(off, C)]], gb[4], sm)
        pltpu.async_copy(ptab_h.at[i5.at[pl.ds(off, C)]], gb[5], sm)
        pltpu.async_copy(box_h.at[pv.at[pl.ds(off, C)]], gb[6], sm)

    def drain(par):
        gb = gsets[par]
        sm = sems[par]
        for j in range(7):
            pltpu.make_async_copy(ptab_h.at[pl.ds(0, C)], gb[j], sm).wait()

    def compute(k, par):
        gb = gsets[par]
        ov = osets[par]

        @pl.when(k >= 2)
        def _():
            pltpu.make_async_copy(
                ov, out_h.at[pl.ds(base + (k - 2) * C, C)],
                osems[par]).wait()

        def token(t, tc):
            s = jnp.zeros((16,), jnp.float32)
            q = jnp.zeros((16,), jnp.float32)
            vs = []
            for c in range(NV):
                sl = pl.ds(c * 16, 16)
                v = (gb[0][t, sl] + gb[1][t, sl] + gb[2][t, sl]
                     + gb[3][t, sl] + gb[4][t, sl] + gb[5][t, sl]
                     + gb[6][t, sl])
                vs.append(v)
                s = s + v
                q = q + v * v
            tot = plsc.cumsum(s)[15]
            totq = plsc.cumsum(q)[15]
            mu = tot * (1.0 / DOUT)
            var = totq * (1.0 / DOUT) - mu * mu
            xv = jnp.full((16,), var + EPS, dtype=jnp.float32)
            iv = lax.bitcast_convert_type(xv, jnp.int32)
            iv = jnp.int32(0x5F3759DF) - (iv >> 1)
            y = lax.bitcast_convert_type(iv, jnp.float32)
            for _ in range(3):
                y = y * (1.5 - 0.5 * xv * y * y)
            muv = jnp.full((16,), mu, dtype=jnp.float32)
            for c in range(NV):
                sl = pl.ds(c * 16, 16)
                ov[t, sl] = (vs[c] - muv) * y * gl[c] + bl[c]
            return tc

        lax.fori_loop(0, C, token, 0, unroll=False)
        pltpu.async_copy(ov, out_h.at[pl.ds(base + k * C, C)], osems[par])

    fire(0, 0)

    def pipe(j, carry):
        k = 2 * j
        fire(k + 1, 1)
        drain(0)
        compute(k, 0)

        @pl.when(k + 2 < NCHUNK)
        def _():
            fire(k + 2, 0)

        drain(1)
        compute(k + 1, 1)
        return carry

    lax.fori_loop(0, NCHUNK // 2, pipe, 0, unroll=False)

    pltpu.make_async_copy(
        ov0, out_h.at[pl.ds(base + (NCHUNK - 2) * C, C)], osem0).wait()
    pltpu.make_async_copy(
        ov1, out_h.at[pl.ds(base + (NCHUNK - 1) * C, C)], osem1).wait()


def kernel(bbox, position_ids, x_tab, y_tab, h_tab, w_tab, box_tab, W, b,
           gamma, beta):
    bb = bbox.reshape(N, 4).astype(jnp.int32)
    x0 = bb[:, 0]
    y0 = bb[:, 1]
    x1 = bb[:, 2]
    y1 = bb[:, 3]
    pos = position_ids.reshape(N).astype(jnp.int32)
    ptab = _project(x_tab, y_tab, h_tab, w_tab, W, b.reshape(1, DOUT))
    out = _lookup(ptab, box_tab, x0, y0, x1, y1, pos, gamma, beta)
    return out.reshape(B, S, DOUT)
